# baseline (speedup 1.0000x reference)
import functools

import jax
import jax.numpy as jnp
from jax import lax
from jax.experimental import pallas as pl
from jax.experimental.pallas import tpu as pltpu
from jax.experimental.pallas import tpu_sc as plsc

N = 10000
D = 128
E = 320000
NC = 2
NS = 16
W = NC * NS
CB = 128
NCHUNK = 80
EP = W * NCHUNK * CB
NBUF = 4
NPAD = 10240
RPT = NPAD // NS

_MESH = plsc.VectorSubcoreMesh(core_axis_name="c", subcore_axis_name="s")


@functools.partial(
    pl.kernel,
    out_type=jax.ShapeDtypeStruct((NC, NPAD), jnp.float32),
    mesh=_MESH,
    scratch_types=[
        pltpu.VMEM((NCHUNK, CB), jnp.int32),
        pltpu.VMEM((CB,), jnp.float32),
        pltpu.VMEM_SHARED((NPAD,), jnp.float32),
    ],
)
def _deg_kernel(to3_hbm, zeros_hbm, ones_hbm, degp_hbm, to_v, ones_v, hist_sh):
    c = lax.axis_index("c")
    s = lax.axis_index("s")
    w = s * NC + c
    pltpu.sync_copy(zeros_hbm, hist_sh.at[pl.ds(s * RPT, RPT)])
    pltpu.sync_copy(ones_hbm, ones_v)
    pltpu.sync_copy(to3_hbm.at[w], to_v)
    plsc.subcore_barrier()

    def body(j, carry):
        pltpu.sync_copy(ones_v, hist_sh.at[to_v.at[j]], add=True)
        return carry

    lax.fori_loop(0, NCHUNK, body, 0)
    plsc.subcore_barrier()
    pltpu.sync_copy(hist_sh.at[pl.ds(s * RPT, RPT)],
                    degp_hbm.at[c, pl.ds(s * RPT, RPT)])


def _scale_body(degp_ref, x_ref, dinv_ref, y_ref):
    deg = degp_ref[0] + degp_ref[1]
    dinv = jnp.where(deg > 0.0, lax.rsqrt(deg), 0.0)
    dinv_ref[...] = dinv
    y_ref[:N] = x_ref[...] * dinv[:N]
    y_ref[N:] = jnp.zeros((NPAD - N, D), jnp.float32)


_scale = pl.pallas_call(
    _scale_body,
    out_shape=(
        jax.ShapeDtypeStruct((NPAD, 1), jnp.float32),
        jax.ShapeDtypeStruct((NPAD, D), jnp.float32),
    ),
)


@functools.partial(
    pl.kernel,
    out_type=jax.ShapeDtypeStruct((NC, NPAD, D), jnp.float32),
    mesh=_MESH,
    scratch_types=[
        pltpu.VMEM((NCHUNK, CB), jnp.int32),
        pltpu.VMEM((NCHUNK, CB), jnp.int32),
        pltpu.VMEM((CB, D), jnp.float32),
        pltpu.VMEM_SHARED((NPAD, D), jnp.float32),
        pltpu.SemaphoreType.DMA,
    ],
)
def _agg_kernel(f3_hbm, t3_hbm, y_hbm, zrows_hbm, accp_hbm,
                f_v, t_v, rows_v, acc_sh, sem):
    c = lax.axis_index("c")
    s = lax.axis_index("s")
    w = s * NC + c
    pltpu.sync_copy(zrows_hbm, acc_sh.at[pl.ds(s * RPT, RPT)])
    pltpu.sync_copy(f3_hbm.at[w], f_v)
    pltpu.sync_copy(t3_hbm.at[w], t_v)
    plsc.subcore_barrier()

    def body(j, carry):
        pltpu.async_copy(y_hbm.at[f_v.at[j]], rows_v, sem).wait()
        pltpu.sync_copy(rows_v, acc_sh.at[t_v.at[j]], add=True)
        return carry

    lax.fori_loop(0, NCHUNK, body, 0)

    plsc.subcore_barrier()
    pltpu.sync_copy(acc_sh.at[pl.ds(s * RPT, RPT)],
                    accp_hbm.at[c, pl.ds(s * RPT, RPT)])


def _combine_body(dinv_ref, accp_ref, out_ref):
    out_ref[...] = dinv_ref[:N] * (accp_ref[0, :N] + accp_ref[1, :N])


_combine = pl.pallas_call(
    _combine_body,
    out_shape=jax.ShapeDtypeStruct((N, D), jnp.float32),
)


def kernel(x, edge_index):
    ei = edge_index.astype(jnp.int32)
    eip = jnp.pad(ei, ((0, 0), (0, EP - E)), constant_values=NPAD - 1)
    f3 = eip[0].reshape(W, NCHUNK, CB)
    t3 = eip[1].reshape(W, NCHUNK, CB)
    zeros_hist = jnp.zeros((RPT,), jnp.float32)
    ones_cb = jnp.ones((CB,), jnp.float32)
    zeros_rows = jnp.zeros((RPT, D), jnp.float32)

    degp = _deg_kernel(t3, zeros_hist, ones_cb)
    dinv, y = _scale(degp.reshape(NC, NPAD, 1), x)
    accp = _agg_kernel(f3, t3, y, zeros_rows)
    return _combine(dinv, accp)


# speedup vs baseline: 2.2508x; 2.2508x over previous
import functools

import jax
import jax.numpy as jnp
from jax import lax
from jax.experimental import pallas as pl
from jax.experimental.pallas import tpu as pltpu
from jax.experimental.pallas import tpu_sc as plsc

N = 10000
D = 128
E = 320000
NC = 2
NS = 16
W = NC * NS
CB = 128
NCHUNK = 80
EP = W * NCHUNK * CB
NBUF = 4
NPAD = 10240
RPT = NPAD // NS

_MESH = plsc.VectorSubcoreMesh(core_axis_name="c", subcore_axis_name="s")


@functools.partial(
    pl.kernel,
    out_type=jax.ShapeDtypeStruct((NC, NPAD), jnp.float32),
    mesh=_MESH,
    scratch_types=[
        pltpu.VMEM((NCHUNK, CB), jnp.int32),
        pltpu.VMEM((CB,), jnp.float32),
        pltpu.VMEM_SHARED((NPAD,), jnp.float32),
    ],
)
def _deg_kernel(to3_hbm, zeros_hbm, ones_hbm, degp_hbm, to_v, ones_v, hist_sh):
    c = lax.axis_index("c")
    s = lax.axis_index("s")
    w = s * NC + c
    pltpu.sync_copy(zeros_hbm, hist_sh.at[pl.ds(s * RPT, RPT)])
    pltpu.sync_copy(ones_hbm, ones_v)
    pltpu.sync_copy(to3_hbm.at[w], to_v)
    plsc.subcore_barrier()

    def body(j, carry):
        pltpu.sync_copy(ones_v, hist_sh.at[to_v.at[j]], add=True)
        return carry

    lax.fori_loop(0, NCHUNK, body, 0)
    plsc.subcore_barrier()
    pltpu.sync_copy(hist_sh.at[pl.ds(s * RPT, RPT)],
                    degp_hbm.at[c, pl.ds(s * RPT, RPT)])


def _scale_body(degp_ref, x_ref, dinv_ref, y_ref):
    deg = degp_ref[0] + degp_ref[1]
    dinv = jnp.where(deg > 0.0, lax.rsqrt(deg), 0.0)
    dinv_ref[...] = dinv
    y_ref[:N] = x_ref[...] * dinv[:N]
    y_ref[N:] = jnp.zeros((NPAD - N, D), jnp.float32)


_scale = pl.pallas_call(
    _scale_body,
    out_shape=(
        jax.ShapeDtypeStruct((NPAD, 1), jnp.float32),
        jax.ShapeDtypeStruct((NPAD, D), jnp.float32),
    ),
)


@functools.partial(
    pl.kernel,
    out_type=jax.ShapeDtypeStruct((NC, NPAD, D), jnp.float32),
    mesh=_MESH,
    scratch_types=[
        pltpu.VMEM((NCHUNK, CB), jnp.int32),
        pltpu.VMEM((NCHUNK, CB), jnp.int32),
        pltpu.VMEM((CB, D), jnp.float32),
        pltpu.VMEM_SHARED((NPAD, D), jnp.float32),
        pltpu.SemaphoreType.DMA,
    ],
)
def _agg_kernel(f3_hbm, t3_hbm, y_hbm, zrows_hbm, accp_hbm,
                f_v, t_v, rows_v, acc_sh, sem):
    c = lax.axis_index("c")
    s = lax.axis_index("s")
    w = s * NC + c
    pltpu.sync_copy(zrows_hbm, acc_sh.at[pl.ds(s * RPT, RPT)])
    pltpu.sync_copy(f3_hbm.at[w], f_v)
    pltpu.sync_copy(t3_hbm.at[w], t_v)
    plsc.subcore_barrier()

    def body(j, carry):
        pltpu.async_copy(y_hbm.at[f_v.at[j]], rows_v, sem).wait()
        pltpu.sync_copy(rows_v, acc_sh.at[t_v.at[j]], add=True)
        return carry

    lax.fori_loop(0, NCHUNK, body, 0)

    plsc.subcore_barrier()
    pltpu.sync_copy(acc_sh.at[pl.ds(s * RPT, RPT)],
                    accp_hbm.at[c, pl.ds(s * RPT, RPT)])


def _combine_body(dinv_ref, accp_ref, out_ref):
    out_ref[...] = dinv_ref[:N] * (accp_ref[0, :N] + accp_ref[1, :N])


_combine = pl.pallas_call(
    _combine_body,
    out_shape=jax.ShapeDtypeStruct((N, D), jnp.float32),
)


def kernel(x, edge_index):
    ei = edge_index.astype(jnp.int32)
    pad_idx = N + (jnp.arange(EP - E, dtype=jnp.int32) % (NPAD - N))
    eip = jnp.concatenate(
        [ei, jnp.stack([pad_idx, pad_idx])], axis=1)
    f3 = eip[0].reshape(W, NCHUNK, CB)
    t3 = eip[1].reshape(W, NCHUNK, CB)
    zeros_hist = jnp.zeros((RPT,), jnp.float32)
    ones_cb = jnp.ones((CB,), jnp.float32)
    zeros_rows = jnp.zeros((RPT, D), jnp.float32)

    degp = _deg_kernel(t3, zeros_hist, ones_cb)
    dinv, y = _scale(degp.reshape(NC, NPAD, 1), x)
    accp = _agg_kernel(f3, t3, y, zeros_rows)
    return _combine(dinv, accp)


# speedup vs baseline: 2.2510x; 1.0001x over previous
import functools

import jax
import jax.numpy as jnp
from jax import lax
from jax.experimental import pallas as pl
from jax.experimental.pallas import tpu as pltpu
from jax.experimental.pallas import tpu_sc as plsc

N = 10000
D = 128
E = 320000
NC = 2
NS = 16
W = NC * NS
CB = 128
NCHUNK = 80
EP = W * NCHUNK * CB
NPAD = 10240
RPT = NPAD // NS

_MESH = plsc.VectorSubcoreMesh(core_axis_name="c", subcore_axis_name="s")


@functools.partial(
    pl.kernel,
    out_type=jax.ShapeDtypeStruct((NC, NPAD), jnp.float32),
    mesh=_MESH,
    scratch_types=[
        pltpu.VMEM((NCHUNK, CB), jnp.int32),
        pltpu.VMEM((CB,), jnp.float32),
        pltpu.VMEM_SHARED((NPAD,), jnp.float32),
    ],
)
def _deg_kernel(to3_hbm, zeros_hbm, ones_hbm, degp_hbm, to_v, ones_v, hist_sh):
    c = lax.axis_index("c")
    s = lax.axis_index("s")
    w = s * NC + c
    pltpu.sync_copy(zeros_hbm, hist_sh.at[pl.ds(s * RPT, RPT)])
    pltpu.sync_copy(ones_hbm, ones_v)
    pltpu.sync_copy(to3_hbm.at[w], to_v)
    plsc.subcore_barrier()

    def body(j, carry):
        pltpu.sync_copy(ones_v, hist_sh.at[to_v.at[j]], add=True)
        return carry

    lax.fori_loop(0, NCHUNK, body, 0)
    plsc.subcore_barrier()
    pltpu.sync_copy(hist_sh.at[pl.ds(s * RPT, RPT)],
                    degp_hbm.at[c, pl.ds(s * RPT, RPT)])


def _scale_body(degp_ref, x_ref, dinv_ref, y_ref):
    deg = degp_ref[0] + degp_ref[1]
    dinv = jnp.where(deg > 0.0, lax.rsqrt(deg), 0.0)
    dinv_ref[...] = dinv
    y_ref[:N] = x_ref[...] * dinv[:N]
    y_ref[N:] = jnp.zeros((NPAD - N, D), jnp.float32)


_scale = pl.pallas_call(
    _scale_body,
    out_shape=(
        jax.ShapeDtypeStruct((NPAD, 1), jnp.float32),
        jax.ShapeDtypeStruct((NPAD, D), jnp.float32),
    ),
)


@functools.partial(
    pl.kernel,
    out_type=jax.ShapeDtypeStruct((NC, NPAD, D), jnp.float32),
    mesh=_MESH,
    scratch_types=[
        pltpu.VMEM((NCHUNK, CB), jnp.int32),
        pltpu.VMEM((NCHUNK, CB), jnp.int32),
        pltpu.VMEM((CB, D), jnp.float32),
        pltpu.VMEM_SHARED((NPAD, D), jnp.float32),
        pltpu.SemaphoreType.DMA,
    ],
)
def _agg_kernel(f3_hbm, t3_hbm, y_hbm, zrows_hbm, accp_hbm,
                f_v, t_v, rows_v, acc_sh, sem):
    c = lax.axis_index("c")
    s = lax.axis_index("s")
    w = s * NC + c
    pltpu.sync_copy(zrows_hbm, acc_sh.at[pl.ds(s * RPT, RPT)])
    pltpu.sync_copy(f3_hbm.at[w], f_v)
    pltpu.sync_copy(t3_hbm.at[w], t_v)
    plsc.subcore_barrier()

    def body(j, carry):
        pltpu.async_copy(y_hbm.at[f_v.at[j]], rows_v, sem).wait()
        pltpu.sync_copy(rows_v, acc_sh.at[t_v.at[j]], add=True)
        return carry

    lax.fori_loop(0, NCHUNK, body, 0)

    plsc.subcore_barrier()
    pltpu.sync_copy(acc_sh.at[pl.ds(s * RPT, RPT)],
                    accp_hbm.at[c, pl.ds(s * RPT, RPT)])


def _combine_body(dinv_ref, accp_ref, out_ref):
    out_ref[...] = dinv_ref[:N] * (accp_ref[0, :N] + accp_ref[1, :N])


_combine = pl.pallas_call(
    _combine_body,
    out_shape=jax.ShapeDtypeStruct((N, D), jnp.float32),
)


def kernel(x, edge_index):
    ei = edge_index.astype(jnp.int32)
    pad_idx = N + (jnp.arange(EP - E, dtype=jnp.int32) % (NPAD - N))
    eip = jnp.concatenate(
        [ei, jnp.stack([pad_idx, pad_idx])], axis=1)
    f3 = eip[0].reshape(W, NCHUNK, CB)
    t3 = eip[1].reshape(W, NCHUNK, CB)
    zeros_hist = jnp.zeros((RPT,), jnp.float32)
    ones_cb = jnp.ones((CB,), jnp.float32)
    zeros_rows = jnp.zeros((RPT, D), jnp.float32)

    degp = _deg_kernel(t3, zeros_hist, ones_cb)
    dinv, y = _scale(degp.reshape(NC, NPAD, 1), x)
    accp = _agg_kernel(f3, t3, y, zeros_rows)
    return _combine(dinv, accp)
